# Initial kernel scaffold; baseline (speedup 1.0000x reference)
import jax
import jax.numpy as jnp
from jax.experimental import pallas as pl


def kernel(anchors, feature_maps):
    raise NotImplementedError("write your pallas kernel here")


# speedup vs baseline: 5.0129x; 5.0129x over previous
import functools

import jax
import jax.numpy as jnp
from jax.experimental import pallas as pl
from jax.experimental.pallas import tpu as pltpu

_B = 2
_A = 20000
_C = 256
_H = 64
_W = 64
_COUNTS = 2000
_AP = 20480
_NS = 32768


def _fmap_integral_kernel(f_ref, out_ref):
    f = f_ref[0]
    s1 = f[:, 0, :]
    for h in range(1, _H):
        s1 = s1 + f[:, h, :]
    y = s1 * jnp.float32(0.015625)
    parts = []
    for j in range(8):
        acc = y[:, j:j + 1]
        for i in range(1, _W // 8):
            acc = acc + y[:, 8 * i + j:8 * i + j + 1]
        parts.append(acc)
    k = 8
    while k > 1:
        h2 = k // 2
        parts = [parts[i] + parts[i + h2] for i in range(h2)]
        k = h2
    m = parts[0] * jnp.float32(0.015625)
    d = jnp.abs(f - m[:, :, None])
    u = d[:128] + d[128:]
    fmap = None
    for j in range(16):
        t0 = (u[8 * j] + u[8 * j + 1]) + (u[8 * j + 2] + u[8 * j + 3])
        t1 = (u[8 * j + 4] + u[8 * j + 5]) + (u[8 * j + 6] + u[8 * j + 7])
        t = t0 + t1
        fmap = t if fmap is None else fmap + t
    rows = [fmap[0:1, :]]
    for r in range(1, _H):
        rows.append(rows[-1] + fmap[r:r + 1, :])
    ii = jnp.concatenate(rows, axis=0)
    cols = [ii[:, 0:1]]
    for c in range(1, _W):
        cols.append(cols[-1] + ii[:, c:c + 1])
    out_ref[0] = jnp.concatenate(cols, axis=1)


def _heat_kernel(x0_ref, y0_ref, x2_ref, y2_ref, ii_ref, heat_ref):
    cum = ii_ref[0]
    nblk = _AP // 512
    for j in range(nblk):
        sl = pl.ds(j * 512, 512)
        x0 = x0_ref[0, 0, sl]
        y0 = y0_ref[0, 0, sl]
        x2 = x2_ref[0, 0, sl]
        y2 = y2_ref[0, 0, sl]
        iota = jax.lax.broadcasted_iota(jnp.int32, (512, _H), 1)

        def row(xi):
            uu = (iota == (xi[:, None] - 1)).astype(jnp.float32)
            return jax.lax.dot_general(
                uu, cum, (((1,), (0,)), ((), ())),
                precision=jax.lax.Precision.HIGHEST)

        def corner(rows, yi):
            v = (iota == (yi[:, None] - 1)).astype(jnp.float32)
            return jnp.sum(rows * v, axis=1)

        r2 = row(x2)
        r0 = row(x0)
        c22 = corner(r2, y2)
        c02 = corner(r0, y2)
        c20 = corner(r2, y0)
        c00 = corner(r0, y0)
        box = ((c22 - c02) - c20) + c00
        numel = ((x2 - x0) * (y2 - y0)).astype(jnp.float32)
        heat = box / numel
        valid = (j * 512 + jax.lax.broadcasted_iota(jnp.int32, (512,), 0)) < _A
        heat_ref[0, 0, sl] = jnp.where(valid, heat, jnp.float32(-1.0))


def _sort_kernel(h_ref, x0_ref, y0_ref, x2_ref, y2_ref,
                 ox0_ref, oy0_ref, ox2_ref, oy2_ref):
    R, L = _NS // 128, 128
    h = h_ref[0]
    planes = [x0_ref[0], y0_ref[0], x2_ref[0], y2_ref[0]]
    ri = jax.lax.broadcasted_iota(jnp.int32, (R, L), 0)
    ci = jax.lax.broadcasted_iota(jnp.int32, (R, L), 1)
    idx = ri * L + ci

    def xchg(x, j):
        if j < L:
            lo = jnp.roll(x, -j, axis=1)
            hi = jnp.roll(x, j, axis=1)
            sel = (ci & j) == 0
        else:
            jr = j // L
            lo = jnp.roll(x, -jr, axis=0)
            hi = jnp.roll(x, jr, axis=0)
            sel = (ri & jr) == 0
        return jnp.where(sel, lo, hi)

    k = 2
    while k <= _NS:
        j = k // 2
        while j >= 1:
            ph = xchg(h, j)
            pidx = xchg(idx, j)
            if j < L:
                is_lower = (ci & j) == 0
            else:
                is_lower = (ri & (j // L)) == 0
            up = (idx & k) == 0
            precedes = (h > ph) | ((h == ph) & (idx < pidx))
            keep = ((up == is_lower) == precedes)
            h = jnp.where(keep, h, ph)
            idx = jnp.where(keep, idx, pidx)
            planes = [jnp.where(keep, p, xchg(p, j)) for p in planes]
            j //= 2
        k *= 2

    ox0_ref[0] = planes[0][:16]
    oy0_ref[0] = planes[1][:16]
    ox2_ref[0] = planes[2][:16]
    oy2_ref[0] = planes[3][:16]


def _plane_pad(x, fill):
    pad = _NS - x.shape[1]
    x = jnp.concatenate(
        [x, jnp.full((_B, pad), fill, x.dtype)], axis=1)
    return x.reshape(_B, _NS // 128, 128)


def kernel(anchors, feature_maps):
    a = anchors[0]
    f = feature_maps[0]
    stride = jnp.array([_H, _W, _H, _W], dtype=jnp.float32)
    anok = jnp.round(a * stride).astype(jnp.int32)

    ii = pl.pallas_call(
        _fmap_integral_kernel,
        grid=(_B,),
        in_specs=[pl.BlockSpec((1, _C, _H, _W), lambda b: (b, 0, 0, 0))],
        out_specs=pl.BlockSpec((1, _H, _W), lambda b: (b, 0, 0)),
        out_shape=jax.ShapeDtypeStruct((_B, _H, _W), jnp.float32),
    )(f)

    pad = _AP - _A
    coord = jnp.transpose(anok, (2, 0, 1))
    padval = jnp.array([0, 0, 1, 1], jnp.int32)[:, None, None]
    coord = jnp.concatenate(
        [coord, jnp.broadcast_to(padval, (4, _B, pad))], axis=2)
    coord = coord[:, :, None, :]

    heats = pl.pallas_call(
        _heat_kernel,
        grid=(_B,),
        in_specs=[pl.BlockSpec((1, 1, _AP), lambda b: (b, 0, 0))] * 4
                 + [pl.BlockSpec((1, _H, _W), lambda b: (b, 0, 0))],
        out_specs=pl.BlockSpec((1, 1, _AP), lambda b: (b, 0, 0)),
        out_shape=jax.ShapeDtypeStruct((_B, 1, _AP), jnp.float32),
    )(coord[0], coord[1], coord[2], coord[3], ii)

    hplane = _plane_pad(heats[:, 0, :], jnp.float32(-1.0))
    aplanes = [_plane_pad(a[:, :, c], jnp.float32(0.0)) for c in range(4)]

    spec = pl.BlockSpec((1, _NS // 128, 128), lambda b: (b, 0, 0))
    ospec = pl.BlockSpec((1, 16, 128), lambda b: (b, 0, 0))
    oshape = jax.ShapeDtypeStruct((_B, 16, 128), jnp.float32)
    outs = pl.pallas_call(
        _sort_kernel,
        grid=(_B,),
        in_specs=[spec] * 5,
        out_specs=[ospec] * 4,
        out_shape=[oshape] * 4,
    )(hplane, *aplanes)

    props = jnp.stack(
        [o.reshape(_B, 16 * 128)[:, :_COUNTS] for o in outs], axis=-1)
    return (props,)
